# Initial kernel scaffold; baseline (speedup 1.0000x reference)
import jax
import jax.numpy as jnp
from jax.experimental import pallas as pl


def kernel(tensor, edge_row, edge_col, edge_val, W):
    raise NotImplementedError("write your pallas kernel here")


# speedup vs baseline: 3.0295x; 3.0295x over previous
import functools

import jax
import jax.numpy as jnp
from jax import lax
from jax.experimental import pallas as pl
from jax.experimental.pallas import tpu as pltpu
from jax.experimental.pallas import tpu_sc as plsc

_N = 10242
_NP = 10752
_RPS = _NP // 16
_E = 71694
_EPC = 4496
_EPAD = 16 * _EPC
_NCHUNK = _EPC // 16
_KS = 20
_NB = 512


def _make_spmv():
    mesh = plsc.VectorSubcoreMesh(core_axis_name="c", subcore_axis_name="s")

    @functools.partial(
        pl.kernel,
        mesh=mesh,
        out_type=jax.ShapeDtypeStruct((2, _NP, 128), jnp.float32),
        scratch_types=[
            pltpu.VMEM((_EPC,), jnp.int32),
            pltpu.VMEM((_EPC,), jnp.int32),
            pltpu.VMEM((_EPC,), jnp.float32),
            pltpu.VMEM((16, 128), jnp.float32),
            pltpu.VMEM_SHARED((_NP, 128), jnp.float32),
            pltpu.SemaphoreType.DMA,
        ],
    )
    def spmv(v_hbm, u_hbm, col_hbm, row_hbm, val_hbm, y_hbm,
             col_v, row_v, val_v, gbuf, acc, sem):
        cid = lax.axis_index("c")
        sid = lax.axis_index("s")
        rbase = sid * _RPS
        pltpu.sync_copy(u_hbm.at[cid, pl.ds(rbase, _RPS)],
                        acc.at[pl.ds(rbase, _RPS)])
        ebase = sid * _EPC
        pltpu.sync_copy(col_hbm.at[pl.ds(ebase, _EPC)], col_v)
        pltpu.sync_copy(row_hbm.at[pl.ds(ebase, _EPC)], row_v)
        pltpu.sync_copy(val_hbm.at[pl.ds(ebase, _EPC)], val_v)
        plsc.subcore_barrier()
        vh = v_hbm.at[cid]

        def body(g, carry):
            base = g * 16
            cidx = col_v[pl.ds(base, 16)]
            pltpu.async_copy(vh.at[cidx], gbuf, sem).wait()
            vchunk = val_v[pl.ds(base, 16)]
            for i in range(16):
                s = vchunk[i]
                for j in range(8):
                    sl = pl.ds(j * 16, 16)
                    gbuf[i, sl] = gbuf[i, sl] * s
            ridx = row_v[pl.ds(base, 16)]
            pltpu.sync_copy(gbuf, acc.at[ridx], add=True)
            return carry

        lax.fori_loop(0, _NCHUNK, body, 0)
        plsc.subcore_barrier()
        pltpu.sync_copy(acc.at[pl.ds(rbase, _RPS)],
                        y_hbm.at[cid, pl.ds(rbase, _RPS)])

    return spmv


_SPMV = _make_spmv()


def _tc_matmul(wbig, zs):
    nblk = _NP // _NB

    def body(wb_ref, *refs):
        z_refs = refs[:_KS]
        out_ref = refs[_KS]
        acc0 = jnp.zeros((_NB, 128), jnp.float32)
        acc1 = jnp.zeros((_NB, 128), jnp.float32)
        for k in range(_KS):
            acc0 += jnp.dot(z_refs[k][0], wb_ref[k],
                            preferred_element_type=jnp.float32)
            acc1 += jnp.dot(z_refs[k][1], wb_ref[k],
                            preferred_element_type=jnp.float32)
        out_ref[0] = acc0
        out_ref[1] = acc1

    return pl.pallas_call(
        body,
        grid=(nblk,),
        in_specs=[pl.BlockSpec((_KS, 128, 128), lambda i: (0, 0, 0))]
        + [pl.BlockSpec((2, _NB, 128), lambda i: (0, i, 0))] * _KS,
        out_specs=pl.BlockSpec((2, _NB, 128), lambda i: (0, i, 0)),
        out_shape=jax.ShapeDtypeStruct((2, _NP, 128), jnp.float32),
    )(wbig, *zs)


def kernel(tensor, edge_row, edge_col, edge_val, W):
    B, T, N, C = tensor.shape
    cout = W.shape[0]
    x = jnp.transpose(tensor, (0, 2, 1, 3)).reshape(2, N, 128)
    x0 = jnp.pad(x, ((0, 0), (0, _NP - N), (0, 0)))
    zero = jnp.zeros((2, _NP, 128), jnp.float32)

    pe = _EPAD - _E
    colp = jnp.concatenate([edge_col, jnp.zeros((pe,), jnp.int32)])
    rowp = jnp.concatenate([edge_row, jnp.zeros((pe,), jnp.int32)])
    valp = jnp.concatenate([edge_val, jnp.zeros((pe,), jnp.float32)])
    val_m2 = -2.0 * valp
    val_p2 = 2.0 * valp

    zs = [x0]
    z1 = _SPMV(x0, zero, colp, rowp, valp)
    zs.append(z1)
    zp0, zp1 = x0, z1
    for k in range(2, _KS):
        vk = val_m2 if k % 2 == 0 else val_p2
        z2 = _SPMV(zp1, zp0, colp, rowp, vk)
        zs.append(z2)
        zp0, zp1 = zp1, z2

    ws = jnp.transpose(W.reshape(cout, C, _KS), (2, 1, 0))
    sgn = jnp.array([1.0 if (k // 2) % 2 == 0 else -1.0 for k in range(_KS)],
                    jnp.float32)
    ws = ws * sgn[:, None, None]
    eye8 = jnp.eye(8, dtype=jnp.float32)
    wbig = jax.vmap(lambda w: jnp.kron(eye8, w))(ws)

    out2 = _tc_matmul(wbig, zs)
    out = out2[:, :N].reshape(2, N, T, cout)
    return jnp.transpose(out, (0, 2, 1, 3)).reshape(B, T, N, cout)
